# baseline (speedup 1.0000x reference)
import functools

import jax
import jax.numpy as jnp
from jax import lax
from jax.experimental import pallas as pl
from jax.experimental.pallas import tpu as pltpu
from jax.experimental.pallas import tpu_sc as plsc

_NUM_CORES = 2
_NUM_SUBCORES = 16
_NUM_WORKERS = _NUM_CORES * _NUM_SUBCORES
_IDX_CHUNK = 128


def _sc_gather_elements(theta_flat, idx3, dim, batch):
    b_per_w = batch // _NUM_WORKERS
    per_d = b_per_w // _IDX_CHUNK
    k_streams = dim * per_d
    mesh = plsc.VectorSubcoreMesh(core_axis_name="c", subcore_axis_name="s")

    @functools.partial(
        pl.kernel,
        mesh=mesh,
        out_type=jax.ShapeDtypeStruct((dim, batch), jnp.float32),
        scratch_types=[
            pltpu.VMEM((k_streams, _IDX_CHUNK), jnp.int32),
            pltpu.VMEM((dim, b_per_w), jnp.float32),
            pltpu.SemaphoreType.DMA,
        ],
    )
    def gather_kernel(flat_hbm, idx_hbm, out_hbm, idx_v, gt_v, sem):
        wid = lax.axis_index("s") * _NUM_CORES + lax.axis_index("c")
        pltpu.sync_copy(idx_hbm.at[wid], idx_v)
        copies = []
        for k in range(k_streams):
            copies.append(
                pltpu.async_copy(
                    flat_hbm.at[idx_v.at[k]],
                    gt_v.at[k // per_d, pl.ds((k % per_d) * _IDX_CHUNK,
                                              _IDX_CHUNK)],
                    sem,
                )
            )
        for c in copies:
            c.wait()
        pltpu.sync_copy(gt_v, out_hbm.at[:, pl.ds(wid * b_per_w, b_per_w)])

    return gather_kernel(theta_flat, idx3)


def _tc_utility_logsoftmax(gt, alpha_item, batch, num_items, dim):
    blk = 1024

    def body(gt_ref, alpha_ref, out_ref):
        g = gt_ref[...]
        u = lax.dot_general(
            g, alpha_ref[...], (((0,), (1,)), ((), ())),
            preferred_element_type=jnp.float32,
        )
        m = jnp.max(u, axis=-1, keepdims=True)
        e = jnp.exp(u - m)
        s = jnp.sum(e, axis=-1, keepdims=True)
        out_ref[...] = u - m - jnp.log(s)

    return pl.pallas_call(
        body,
        grid=(batch // blk,),
        in_specs=[
            pl.BlockSpec((dim, blk), lambda i: (0, i)),
            pl.BlockSpec((num_items, dim), lambda i: (0, 0)),
        ],
        out_specs=pl.BlockSpec((blk, num_items), lambda i: (i, 0)),
        out_shape=jax.ShapeDtypeStruct((batch, num_items), jnp.float32),
    )(gt, alpha_item)


def kernel(user_index, theta_user, alpha_item):
    batch = user_index.shape[0]
    num_users, dim = theta_user.shape
    num_items = alpha_item.shape[0]
    b_per_w = batch // _NUM_WORKERS

    idx = user_index.astype(jnp.int32)
    u_r = idx.reshape(_NUM_WORKERS, 1, b_per_w)
    d_off = (jnp.arange(dim, dtype=jnp.int32) * num_users).reshape(1, dim, 1)
    idx3 = (u_r + d_off).reshape(
        _NUM_WORKERS, dim * b_per_w // _IDX_CHUNK, _IDX_CHUNK)

    theta_flat = theta_user.T.reshape(num_users * dim)
    gt = _sc_gather_elements(theta_flat, idx3, dim, batch)
    return _tc_utility_logsoftmax(gt, alpha_item, batch, num_items, dim)
